# baseline (device time: 17190 ns/iter reference)
import jax
import jax.numpy as jnp
from jax import lax
from jax.experimental import pallas as pl
from jax.experimental.pallas import tpu as pltpu

Z = 4


def kernel(x, W, labels):
    t, d = x.shape
    v_local = W.shape[1]

    def body(x_ref, w_ref, labels_ref, out_ref, gbuf, send_sems, recv_sems):
        my_x = lax.axis_index("x")
        my_y = lax.axis_index("y")
        my_z = lax.axis_index("z")
        left = lax.rem(my_z - 1 + Z, Z)
        right = lax.rem(my_z + 1, Z)

        barrier_sem = pltpu.get_barrier_semaphore()
        for nbr in (left, right):
            pl.semaphore_signal(
                barrier_sem,
                inc=1,
                device_id=(my_x, my_y, nbr),
                device_id_type=pl.DeviceIdType.MESH,
            )
        pl.semaphore_wait(barrier_sem, 2)

        logits = jnp.dot(
            x_ref[:, :], w_ref[:, :], preferred_element_type=jnp.float32
        )
        m = jnp.max(logits, axis=1)
        s = jnp.sum(jnp.exp(logits - m[:, None]), axis=1)
        col = lax.broadcasted_iota(jnp.int32, (t, v_local), 1) + my_z * v_local
        mask = col == labels_ref[:].reshape(t, 1)
        lab = jnp.sum(jnp.where(mask, logits, 0.0), axis=1)

        gbuf[my_z] = jnp.stack([m, s, lab])

        for h in range(Z - 1):
            origin_send = lax.rem(my_z - h + Z, Z)
            rdma = pltpu.make_async_remote_copy(
                src_ref=gbuf.at[origin_send],
                dst_ref=gbuf.at[origin_send],
                send_sem=send_sems.at[h],
                recv_sem=recv_sems.at[h],
                device_id=(my_x, my_y, right),
                device_id_type=pl.DeviceIdType.MESH,
            )
            rdma.start()
            rdma.wait()

        ms = gbuf[:, 0, :]
        ss = gbuf[:, 1, :]
        labs = gbuf[:, 2, :]
        gmax = jnp.max(ms, axis=0)
        gsum = jnp.sum(ss * jnp.exp(ms - gmax[None, :]), axis=0)
        glab = jnp.sum(labs, axis=0)
        out_ref[:] = gmax + jnp.log(gsum) - glab

    return pl.pallas_call(
        body,
        out_shape=jax.ShapeDtypeStruct((t,), jnp.float32),
        in_specs=[
            pl.BlockSpec(memory_space=pltpu.VMEM),
            pl.BlockSpec(memory_space=pltpu.VMEM),
            pl.BlockSpec(memory_space=pltpu.VMEM),
        ],
        out_specs=pl.BlockSpec(memory_space=pltpu.VMEM),
        scratch_shapes=[
            pltpu.VMEM((Z, 3, t), jnp.float32),
            pltpu.SemaphoreType.DMA((Z - 1,)),
            pltpu.SemaphoreType.DMA((Z - 1,)),
        ],
        compiler_params=pltpu.CompilerParams(collective_id=0),
    )(x, W, labels)


# device time: 13450 ns/iter; 1.2781x vs baseline; 1.2781x over previous
import jax
import jax.numpy as jnp
from jax import lax
from jax.experimental import pallas as pl
from jax.experimental.pallas import tpu as pltpu

Z = 4


def kernel(x, W, labels):
    t, d = x.shape
    v_local = W.shape[1]

    def body(x_ref, w_ref, labels_ref, out_ref, gbuf, send_sems, recv_sems):
        my_x = lax.axis_index("x")
        my_y = lax.axis_index("y")
        my_z = lax.axis_index("z")

        barrier_sem = pltpu.get_barrier_semaphore()
        for dz in range(1, Z):
            peer = lax.rem(my_z + dz, Z)
            pl.semaphore_signal(
                barrier_sem,
                inc=1,
                device_id=(my_x, my_y, peer),
                device_id_type=pl.DeviceIdType.MESH,
            )

        logits = jnp.dot(
            x_ref[:, :].astype(jnp.bfloat16),
            w_ref[:, :].astype(jnp.bfloat16),
            preferred_element_type=jnp.float32,
        )
        m = jnp.max(logits, axis=1)
        s = jnp.sum(jnp.exp(logits - m[:, None]), axis=1)
        col = lax.broadcasted_iota(jnp.int32, (t, v_local), 1) + my_z * v_local
        mask = col == labels_ref[:].reshape(t, 1)
        lab = jnp.sum(jnp.where(mask, logits, 0.0), axis=1)

        gbuf[my_z] = jnp.stack([m, s, lab])

        pl.semaphore_wait(barrier_sem, Z - 1)

        sends = []
        for dz in range(1, Z):
            peer = lax.rem(my_z + dz, Z)
            rdma = pltpu.make_async_remote_copy(
                src_ref=gbuf.at[my_z],
                dst_ref=gbuf.at[my_z],
                send_sem=send_sems.at[dz - 1],
                recv_sem=recv_sems.at[dz - 1],
                device_id=(my_x, my_y, peer),
                device_id_type=pl.DeviceIdType.MESH,
            )
            rdma.start()
            sends.append(rdma)

        for dz in range(1, Z):
            origin = lax.rem(my_z - dz + Z, Z)
            recv = pltpu.make_async_remote_copy(
                src_ref=gbuf.at[origin],
                dst_ref=gbuf.at[origin],
                send_sem=send_sems.at[dz - 1],
                recv_sem=recv_sems.at[dz - 1],
                device_id=(my_x, my_y, my_z),
                device_id_type=pl.DeviceIdType.MESH,
            )
            recv.wait_recv()

        ms = gbuf[:, 0, :]
        ss = gbuf[:, 1, :]
        labs = gbuf[:, 2, :]
        gmax = jnp.max(ms, axis=0)
        gsum = jnp.sum(ss * jnp.exp(ms - gmax[None, :]), axis=0)
        glab = jnp.sum(labs, axis=0)
        out_ref[:] = gmax + jnp.log(gsum) - glab

        for rdma in sends:
            rdma.wait_send()

    return pl.pallas_call(
        body,
        out_shape=jax.ShapeDtypeStruct((t,), jnp.float32),
        in_specs=[
            pl.BlockSpec(memory_space=pltpu.VMEM),
            pl.BlockSpec(memory_space=pltpu.VMEM),
            pl.BlockSpec(memory_space=pltpu.VMEM),
        ],
        out_specs=pl.BlockSpec(memory_space=pltpu.VMEM),
        scratch_shapes=[
            pltpu.VMEM((Z, 3, t), jnp.float32),
            pltpu.SemaphoreType.DMA((Z - 1,)),
            pltpu.SemaphoreType.DMA((Z - 1,)),
        ],
        compiler_params=pltpu.CompilerParams(collective_id=0),
    )(x, W, labels)
